# baseline (speedup 1.0000x reference)
import functools
import math

import jax
import jax.numpy as jnp
from jax import lax
from jax.experimental import pallas as pl
from jax.experimental.pallas import tpu as pltpu
from jax.experimental.pallas import tpu_sc as plsc

D_MODEL = 64
SCALE = math.sqrt(D_MODEL)

NC = 2
NS = 16
NW = NC * NS
LANES = 16

SPG = 4
GBUF = 2
TBUF = 4


def _make_emb_kernel(NR, NB):
  mesh = plsc.VectorSubcoreMesh(core_axis_name="c", subcore_axis_name="s")
  n_units = NR * 2
  n_blocks = NR * 8

  @functools.partial(
      pl.kernel,
      out_type=jax.ShapeDtypeStruct((n_blocks, 8, NB, 8, 128), jnp.float32),
      mesh=mesh,
      scratch_types=(
          [pltpu.VMEM((512,), jnp.int32) for _ in range(4)]
          + [pltpu.VMEM((SPG * 128, D_MODEL), jnp.float32)
             for _ in range(GBUF)]
          + [pltpu.VMEM((8, 8, 128), jnp.float32) for _ in range(TBUF)]
          + [
              pltpu.SemaphoreType.DMA((2,)),
              pltpu.SemaphoreType.DMA((GBUF,)),
              pltpu.SemaphoreType.DMA((TBUF,)),
          ]
      ),
      compiler_params=pltpu.CompilerParams(
          use_tc_tiling_on_sc=False, needs_layout_passes=False
      ),
  )
  def emb(idx4_hbm, table_hbm, out5_hbm, ib0, ib1, ib2, ib3, rb0, rb1,
          tb0, tb1, tb2, tb3, isem, gsem, osem):
    ibufs = (ib0, ib1, ib2, ib3)
    rbufs = (rb0, rb1)
    tbufs = (tb0, tb1, tb2, tb3)
    w = lax.axis_index("s") * NC + lax.axis_index("c")
    lane = lax.iota(jnp.int32, LANES)

    def issue_idx(R, slot):
      for h in range(2):
        pltpu.async_copy(idx4_hbm.at[R, w, pl.ds(h * 512, 512)],
                         ibufs[slot * 2 + h], isem.at[slot])

    def wait_idx(R, slot):
      for h in range(2):
        pltpu.make_async_copy(idx4_hbm.at[R, w, pl.ds(h * 512, 512)],
                              ibufs[slot * 2 + h], isem.at[slot]).wait()

    def issue_gather(half, islot, slot):
      pltpu.async_copy(table_hbm.at[ibufs[islot * 2 + half]],
                       rbufs[slot], gsem.at[slot])

    def wait_gather(half, islot, slot):
      pltpu.make_async_copy(table_hbm.at[ibufs[islot * 2 + half]],
                            rbufs[slot], gsem.at[slot]).wait()

    def issue_out(r, slot):
      pltpu.async_copy(tbufs[slot], out5_hbm.at[r, :, w], osem.at[slot])

    def wait_out(r, slot):
      pltpu.make_async_copy(tbufs[slot], out5_hbm.at[r, :, w],
                            osem.at[slot]).wait()

    issue_idx(0, 0)
    issue_idx(1, 1)
    wait_idx(0, 0)
    issue_gather(0, 0, 0)

    def unit(u, carry):
      R = u // 2
      half = u % 2

      @pl.when(u + 1 < n_units)
      def _():
        u1 = u + 1
        for i2 in range(2):
          @pl.when(jnp.equal((u1 // 2) % 2, i2))
          def _():
            @pl.when(jnp.equal(u1 % 2, 0))
            def _():
              wait_idx(u1 // 2, i2)

            for h2 in range(2):
              @pl.when(jnp.equal(u1 % 2, h2))
              def _():
                for g in range(GBUF):
                  @pl.when(jnp.equal(u1 % GBUF, g))
                  def _():
                    issue_gather(h2, i2, g)

      for g in range(GBUF):
        @pl.when(jnp.equal(u % GBUF, g))
        def _():
          for i2 in range(2):
            @pl.when(jnp.equal(R % 2, i2))
            def _():
              for h2 in range(2):
                @pl.when(jnp.equal(half, h2))
                def _():
                  wait_gather(h2, i2, g)

              @pl.when((half == 1) & (R + 2 < NR))
              def _():
                issue_idx(R + 2, i2)

          rb = rbufs[g]
          for k in range(SPG):
            tb = tbufs[k]
            r = R * 8 + half * SPG + k

            @pl.when(u >= 1)
            def _():
              wait_out(r - TBUF, k)

            @plsc.parallel_loop(0, 8, 1, unroll=2)
            def _xpose(lgrp):
              rows = k * 128 + lgrp * LANES + lane
              sl = pl.ds(lgrp * LANES, LANES)
              for p in range(8):
                vals = [
                    plsc.load_gather(
                        rb, [rows, jnp.full((LANES,), p * 8 + q, jnp.int32)]
                    )
                    for q in range(8)
                ]
                for q in range(8):
                  tb[p, q, sl] = vals[q] * SCALE

            issue_out(r, k)

      return carry

    lax.fori_loop(0, n_units, unit, 0)

    for r in range(n_blocks - TBUF, n_blocks):
      wait_out(r, r % TBUF)

  return emb


def kernel(x, lut):
  NB, NR = x.shape[0] // 128, x.shape[1] // 8
  idx4 = (
      x.astype(jnp.int32)
      .reshape(NB, 128, NR, 8)
      .transpose(2, 0, 3, 1)
      .reshape(NR, NB, 1024)
  )
  out5 = _make_emb_kernel(NR, NB)(idx4, lut)
  return (
      out5.reshape(NR * 8, 8, NB, 8, 128)
      .transpose(2, 4, 0, 1, 3)
      .reshape(x.shape[0], x.shape[1], D_MODEL)
  )


# speedup vs baseline: 1.6048x; 1.6048x over previous
import functools
import math

import jax
import jax.numpy as jnp
from jax import lax
from jax.experimental import pallas as pl
from jax.experimental.pallas import tpu as pltpu
from jax.experimental.pallas import tpu_sc as plsc

D_MODEL = 64
SCALE = math.sqrt(D_MODEL)

NC = 2
NS = 16
NW = NC * NS
LANES = 16

SPG = 4
GBUF = 2
TBUF = 4


def _make_emb_kernel(NR, NB):
  mesh = plsc.VectorSubcoreMesh(core_axis_name="c", subcore_axis_name="s")
  n_units = NR * 2
  n_blocks = NR * 8

  @functools.partial(
      pl.kernel,
      out_type=jax.ShapeDtypeStruct((n_blocks, 8, NB, 8, 128), jnp.float32),
      mesh=mesh,
      scratch_types=(
          [pltpu.VMEM((512,), jnp.int32) for _ in range(4)]
          + [pltpu.VMEM((SPG * 128, D_MODEL), jnp.float32)
             for _ in range(GBUF)]
          + [pltpu.VMEM((8, 8, 128), jnp.float32) for _ in range(TBUF)]
          + [pltpu.VMEM((D_MODEL, 129), jnp.float32)]
          + [
              pltpu.SemaphoreType.DMA((2,)),
              pltpu.SemaphoreType.DMA((GBUF,)),
              pltpu.SemaphoreType.DMA((TBUF,)),
          ]
      ),
      compiler_params=pltpu.CompilerParams(
          use_tc_tiling_on_sc=False, needs_layout_passes=False
      ),
  )
  def emb(idx4_hbm, table_hbm, out5_hbm, ib0, ib1, ib2, ib3, rb0, rb1,
          tb0, tb1, tb2, tb3, tbp, isem, gsem, osem):
    ibufs = (ib0, ib1, ib2, ib3)
    rbufs = (rb0, rb1)
    tbufs = (tb0, tb1, tb2, tb3)
    w = lax.axis_index("s") * NC + lax.axis_index("c")
    lane = lax.iota(jnp.int32, LANES)

    def issue_idx(R, slot):
      for h in range(2):
        pltpu.async_copy(idx4_hbm.at[R, w, pl.ds(h * 512, 512)],
                         ibufs[slot * 2 + h], isem.at[slot])

    def wait_idx(R, slot):
      for h in range(2):
        pltpu.make_async_copy(idx4_hbm.at[R, w, pl.ds(h * 512, 512)],
                              ibufs[slot * 2 + h], isem.at[slot]).wait()

    def issue_gather(half, islot, slot):
      pltpu.async_copy(table_hbm.at[ibufs[islot * 2 + half]],
                       rbufs[slot], gsem.at[slot])

    def wait_gather(half, islot, slot):
      pltpu.make_async_copy(table_hbm.at[ibufs[islot * 2 + half]],
                            rbufs[slot], gsem.at[slot]).wait()

    def issue_out(r, slot):
      pltpu.async_copy(tbufs[slot], out5_hbm.at[r, :, w], osem.at[slot])

    def wait_out(r, slot):
      pltpu.make_async_copy(tbufs[slot], out5_hbm.at[r, :, w],
                            osem.at[slot]).wait()

    issue_idx(0, 0)
    issue_idx(1, 1)
    wait_idx(0, 0)
    issue_gather(0, 0, 0)

    def unit(u, carry):
      R = u // 2
      half = u % 2

      @pl.when(u + 1 < n_units)
      def _():
        u1 = u + 1
        for i2 in range(2):
          @pl.when(jnp.equal((u1 // 2) % 2, i2))
          def _():
            @pl.when(jnp.equal(u1 % 2, 0))
            def _():
              wait_idx(u1 // 2, i2)

            for h2 in range(2):
              @pl.when(jnp.equal(u1 % 2, h2))
              def _():
                for g in range(GBUF):
                  @pl.when(jnp.equal(u1 % GBUF, g))
                  def _():
                    issue_gather(h2, i2, g)

      for g in range(GBUF):
        @pl.when(jnp.equal(u % GBUF, g))
        def _():
          for i2 in range(2):
            @pl.when(jnp.equal(R % 2, i2))
            def _():
              for h2 in range(2):
                @pl.when(jnp.equal(half, h2))
                def _():
                  wait_gather(h2, i2, g)

              @pl.when((half == 1) & (R + 2 < NR))
              def _():
                issue_idx(R + 2, i2)

          rb = rbufs[g]
          for k in range(SPG):
            tb = tbufs[k]
            r = R * 8 + half * SPG + k

            @pl.when(u >= 1)
            def _():
              wait_out(r - TBUF, k)

            @plsc.parallel_loop(0, 128, 1, unroll=4)
            def _fill(l):
              cols = jnp.full((LANES,), l, jnp.int32)
              for c in range(4):
                vals = rb[k * 128 + l, pl.ds(c * LANES, LANES)]
                plsc.store_scatter(tbp, [c * LANES + lane, cols], vals)

            @plsc.parallel_loop(0, 8, 1, unroll=2)
            def _emit(lgrp):
              sl = pl.ds(lgrp * LANES, LANES)
              for p in range(8):
                for q in range(8):
                  tb[p, q, sl] = tbp[p * 8 + q, sl] * SCALE

            issue_out(r, k)

      return carry

    lax.fori_loop(0, n_units, unit, 0)

    for r in range(n_blocks - TBUF, n_blocks):
      wait_out(r, r % TBUF)

  return emb


def kernel(x, lut):
  NB, NR = x.shape[0] // 128, x.shape[1] // 8
  idx4 = (
      x.astype(jnp.int32)
      .reshape(NB, 128, NR, 8)
      .transpose(2, 0, 3, 1)
      .reshape(NR, NB, 1024)
  )
  out5 = _make_emb_kernel(NR, NB)(idx4, lut)
  return (
      out5.reshape(NR * 8, 8, NB, 8, 128)
      .transpose(2, 4, 0, 1, 3)
      .reshape(x.shape[0], x.shape[1], D_MODEL)
  )
